# Initial kernel scaffold; baseline (speedup 1.0000x reference)
import functools

import jax
import jax.numpy as jnp
from jax import lax
from jax.experimental import pallas as pl
from jax.experimental.pallas import tpu as pltpu
from jax.experimental.pallas import tpu_sc as plsc

N = 10000
E = 320000
K = 25
NG = 128

NC = 2
NS = 16
NW = NC * NS
EW = E // NW
CH = 80
NCH = EW // CH

_MESH = dict(core_axis_name="c", subcore_axis_name="s")


def _sc_gather(table, idx, d):
    @functools.partial(
        pl.kernel,
        mesh=plsc.VectorSubcoreMesh(**_MESH),
        out_type=jax.ShapeDtypeStruct((E, d), jnp.float32),
        scratch_types=[
            pltpu.VMEM((CH,), jnp.int32),
            pltpu.VMEM((CH, d), jnp.float32),
            pltpu.SemaphoreType.DMA,
        ],
    )
    def k(table_hbm, idx_hbm, out_hbm, idx_v, rows_v, sem):
        wid = lax.axis_index("s") * NC + lax.axis_index("c")
        base = wid * EW

        @pl.loop(0, NCH)
        def _(c):
            off = base + c * CH
            pltpu.sync_copy(idx_hbm.at[pl.ds(off, CH)], idx_v)
            pltpu.async_copy(table_hbm.at[idx_v], rows_v, sem).wait()
            pltpu.sync_copy(rows_v, out_hbm.at[pl.ds(off, CH)])

    return k(table, idx)


def _sc_scatter_add(vals, idx, d):
    @functools.partial(
        pl.kernel,
        mesh=plsc.VectorSubcoreMesh(**_MESH),
        out_type=jax.ShapeDtypeStruct((NC, N, d), jnp.float32),
        scratch_types=[
            pltpu.VMEM_SHARED((N, d), jnp.float32),
            pltpu.VMEM((CH,), jnp.int32),
            pltpu.VMEM((CH, d), jnp.float32),
            pltpu.SemaphoreType.DMA,
        ],
    )
    def k(vals_hbm, idx_hbm, zero_hbm, out_hbm, acc_s, idx_v, val_v, sem):
        cid = lax.axis_index("c")
        sid = lax.axis_index("s")

        @pl.when(sid == 0)
        def _():
            pltpu.sync_copy(zero_hbm, acc_s)

        plsc.subcore_barrier()
        base = (sid * NC + cid) * EW

        @pl.loop(0, NCH)
        def _(c):
            off = base + c * CH
            pltpu.sync_copy(idx_hbm.at[pl.ds(off, CH)], idx_v)
            pltpu.sync_copy(vals_hbm.at[pl.ds(off, CH)], val_v)
            pltpu.sync_copy(val_v, acc_s.at[idx_v], add=True)

        plsc.subcore_barrier()

        @pl.when(sid == 0)
        def _():
            pltpu.sync_copy(acc_s, out_hbm.at[cid])

    zeros = jnp.zeros((N, d), jnp.float32)
    return k(vals, idx, zeros)


def _sc_count(idx):
    @functools.partial(
        pl.kernel,
        mesh=plsc.VectorSubcoreMesh(**_MESH),
        out_type=jax.ShapeDtypeStruct((NC, N, 16), jnp.float32),
        scratch_types=[
            pltpu.VMEM_SHARED((N, 16), jnp.float32),
            pltpu.VMEM((CH,), jnp.int32),
            pltpu.VMEM((CH, 16), jnp.float32),
            pltpu.SemaphoreType.DMA,
        ],
    )
    def k(idx_hbm, ones_hbm, zero_hbm, out_hbm, acc_s, idx_v, val_v, sem):
        cid = lax.axis_index("c")
        sid = lax.axis_index("s")
        pltpu.sync_copy(ones_hbm, val_v)

        @pl.when(sid == 0)
        def _():
            pltpu.sync_copy(zero_hbm, acc_s)

        plsc.subcore_barrier()
        base = (sid * NC + cid) * EW

        @pl.loop(0, NCH)
        def _(c):
            pltpu.sync_copy(idx_hbm.at[pl.ds(base + c * CH, CH)], idx_v)
            pltpu.sync_copy(val_v, acc_s.at[idx_v], add=True)

        plsc.subcore_barrier()

        @pl.when(sid == 0)
        def _():
            pltpu.sync_copy(acc_s, out_hbm.at[cid])

    ones = jnp.ones((CH, 16), jnp.float32)
    zeros = jnp.zeros((N, 16), jnp.float32)
    return k(idx, ones, zeros)


BE = 512


def _tc_edge(xe, attr, g, mu0, mu1, s0, s1, ci, co):
    def body(xe_ref, attr_ref, g_ref, mu0_ref, mu1_ref, s0_ref, s1_ref, o_ref):
        h = jnp.dot(xe_ref[...], g_ref[...], preferred_element_type=jnp.float32)
        a = attr_ref[...]
        a0 = a[:, 0:1]
        a1 = a[:, 1:2]
        r0 = 1.0 / (1e-14 + s0_ref[...] * s0_ref[...])
        r1 = 1.0 / (1e-14 + s1_ref[...] * s1_ref[...])
        d0 = a0 - mu0_ref[...]
        d1 = a1 - mu1_ref[...]
        w = jnp.exp(-0.5 * (d0 * d0 * r0 + d1 * d1 * r1))
        acc = w[:, 0:1] * h[:, 0:co]
        for kk in range(1, K):
            acc = acc + w[:, kk:kk + 1] * h[:, kk * co:(kk + 1) * co]
        o_ref[...] = acc

    return pl.pallas_call(
        body,
        grid=(E // BE,),
        in_specs=[
            pl.BlockSpec((BE, ci), lambda i: (i, 0)),
            pl.BlockSpec((BE, 2), lambda i: (i, 0)),
            pl.BlockSpec((ci, K * co), lambda i: (0, 0)),
            pl.BlockSpec((1, K), lambda i: (0, 0)),
            pl.BlockSpec((1, K), lambda i: (0, 0)),
            pl.BlockSpec((1, K), lambda i: (0, 0)),
            pl.BlockSpec((1, K), lambda i: (0, 0)),
        ],
        out_specs=pl.BlockSpec((BE, co), lambda i: (i, 0)),
        out_shape=jax.ShapeDtypeStruct((E, co), jnp.float32),
    )(xe, attr, g, mu0, mu1, s0, s1)


def _tc_epilogue(parts, cnt_parts, x_in, root, bias, bn_g, bn_b, ci, co):
    def body(p_ref, c_ref, x_ref, r_ref, b_ref, g_ref, be_ref, o_ref):
        ssum = p_ref[0] + p_ref[1]
        cnt = c_ref[0, :, 0:1] + c_ref[1, :, 0:1]
        agg = ssum / jnp.maximum(cnt, 1.0)
        o = agg + jnp.dot(x_ref[...], r_ref[...],
                          preferred_element_type=jnp.float32) + b_ref[...]
        m = jnp.mean(o, axis=0, keepdims=True)
        v = jnp.mean((o - m) * (o - m), axis=0, keepdims=True)
        o = (o - m) / jnp.sqrt(v + 1e-5) * g_ref[...] + be_ref[...]
        o_ref[...] = jnp.where(o > 0.0, o, jnp.exp(o) - 1.0)

    return pl.pallas_call(
        body,
        out_shape=jax.ShapeDtypeStruct((N, co), jnp.float32),
    )(parts, cnt_parts, x_in, root, bias.reshape(1, co),
      bn_g.reshape(1, co), bn_b.reshape(1, co))


def _tc_final(h, batch2d, fc1_w, fc1_b, fc2_w, fc2_b):
    def body(h_ref, b_ref, w1_ref, b1_ref, w2_ref, b2_ref, o_ref):
        oh = (b_ref[...] == lax.broadcasted_iota(jnp.int32, (N, NG), 1))
        oh = oh.astype(jnp.float32)
        pooled = lax.dot_general(oh, h_ref[...], (((0,), (0,)), ((), ())),
                                 preferred_element_type=jnp.float32)
        ones = jnp.full((N, 1), 1.0, jnp.float32)
        pcnt = lax.dot_general(oh, ones, (((0,), (0,)), ((), ())),
                               preferred_element_type=jnp.float32)
        pooled = pooled / jnp.maximum(pcnt, 1.0)
        t = jnp.dot(pooled, w1_ref[...],
                    preferred_element_type=jnp.float32) + b1_ref[...]
        t = jnp.where(t > 0.0, t, jnp.exp(t) - 1.0)
        t = jnp.dot(t, w2_ref[...],
                    preferred_element_type=jnp.float32) + b2_ref[...]
        t = jnp.where(t > 0.0, t, jnp.exp(t) - 1.0)
        m = jnp.max(t, axis=1, keepdims=True)
        s = jnp.log(jnp.sum(jnp.exp(t - m), axis=1, keepdims=True))
        o_ref[...] = t - m - s

    return pl.pallas_call(
        body,
        out_shape=jax.ShapeDtypeStruct((NG, 10), jnp.float32),
    )(h, batch2d, fc1_w, fc1_b.reshape(1, -1), fc2_w, fc2_b.reshape(1, -1))


def kernel(x, edge_index, edge_attr, batch,
           g1, mu1, sigma1, root1, bias1, bn1_g, bn1_b,
           g2, mu2, sigma2, root2, bias2, bn2_g, bn2_b,
           g3, mu3, sigma3, root3, bias3, bn3_g, bn3_b,
           fc1_w, fc1_b, fc2_w, fc2_b):
    src = edge_index[0]
    dst = edge_index[1]
    cnt_parts = _sc_count(dst)

    layers = [
        (g1, mu1, sigma1, root1, bias1, bn1_g, bn1_b, 128, 32),
        (g2, mu2, sigma2, root2, bias2, bn2_g, bn2_b, 32, 64),
        (g3, mu3, sigma3, root3, bias3, bn3_g, bn3_b, 64, 64),
    ]
    h = x
    for (g, mu, sigma, root, bias, bng, bnb, ci, co) in layers:
        xe = _sc_gather(h, src, ci)
        msg = _tc_edge(xe, edge_attr, g,
                       mu[:, 0].reshape(1, K), mu[:, 1].reshape(1, K),
                       sigma[:, 0].reshape(1, K), sigma[:, 1].reshape(1, K),
                       ci, co)
        parts = _sc_scatter_add(msg, dst, co)
        h = _tc_epilogue(parts, cnt_parts, h, root, bias, bng, bnb, ci, co)

    return _tc_final(h, batch.reshape(N, 1), fc1_w, fc1_b, fc2_w, fc2_b)


# speedup vs baseline: 1.0724x; 1.0724x over previous
import functools

import jax
import jax.numpy as jnp
from jax import lax
from jax.experimental import pallas as pl
from jax.experimental.pallas import tpu as pltpu
from jax.experimental.pallas import tpu_sc as plsc

N = 10000
E = 320000
K = 25
NG = 128

NC = 2
NS = 16
NW = NC * NS
EW = E // NW
CH = 80
NCH = EW // CH

_MESH = dict(core_axis_name="c", subcore_axis_name="s")


def _sc_gather(table, idx, d):
    @functools.partial(
        pl.kernel,
        mesh=plsc.VectorSubcoreMesh(**_MESH),
        out_type=jax.ShapeDtypeStruct((E, d), jnp.float32),
        scratch_types=[
            pltpu.VMEM((CH,), jnp.int32),
            pltpu.VMEM((CH, d), jnp.float32),
            pltpu.SemaphoreType.DMA,
        ],
    )
    def k(table_hbm, idx_hbm, out_hbm, idx_v, rows_v, sem):
        wid = lax.axis_index("s") * NC + lax.axis_index("c")
        base = wid * EW

        @pl.loop(0, NCH)
        def _(c):
            off = base + c * CH
            pltpu.sync_copy(idx_hbm.at[pl.ds(off, CH)], idx_v)
            pltpu.async_copy(table_hbm.at[idx_v], rows_v, sem).wait()
            pltpu.sync_copy(rows_v, out_hbm.at[pl.ds(off, CH)])

    return k(table, idx)


def _sc_scatter_add(vals, idx, d):
    @functools.partial(
        pl.kernel,
        mesh=plsc.VectorSubcoreMesh(**_MESH),
        out_type=jax.ShapeDtypeStruct((NC, N, d), jnp.float32),
        scratch_types=[
            pltpu.VMEM_SHARED((N, d), jnp.float32),
            pltpu.VMEM((CH,), jnp.int32),
            pltpu.VMEM((CH, d), jnp.float32),
            pltpu.SemaphoreType.DMA,
        ],
    )
    def k(vals_hbm, idx_hbm, zero_hbm, out_hbm, acc_s, idx_v, val_v, sem):
        cid = lax.axis_index("c")
        sid = lax.axis_index("s")

        @pl.when(sid == 0)
        def _():
            pltpu.sync_copy(zero_hbm, acc_s)

        plsc.subcore_barrier()
        base = (sid * NC + cid) * EW

        @pl.loop(0, NCH)
        def _(c):
            off = base + c * CH
            pltpu.sync_copy(idx_hbm.at[pl.ds(off, CH)], idx_v)
            pltpu.sync_copy(vals_hbm.at[pl.ds(off, CH)], val_v)
            pltpu.sync_copy(val_v, acc_s.at[idx_v], add=True)

        plsc.subcore_barrier()

        @pl.when(sid == 0)
        def _():
            pltpu.sync_copy(acc_s, out_hbm.at[cid])

    zeros = jnp.zeros((N, d), jnp.float32)
    return k(vals, idx, zeros)


BE = 512


def _tc_edge(xe, attr, g, mu0, mu1, s0, s1, ci, co):
    def body(xe_ref, attr_ref, g_ref, mu0_ref, mu1_ref, s0_ref, s1_ref, o_ref):
        h = jnp.dot(xe_ref[...][:, :ci], g_ref[...],
                    preferred_element_type=jnp.float32)
        a = attr_ref[...]
        a0 = a[:, 0:1]
        a1 = a[:, 1:2]
        r0 = 1.0 / (1e-14 + s0_ref[...] * s0_ref[...])
        r1 = 1.0 / (1e-14 + s1_ref[...] * s1_ref[...])
        d0 = a0 - mu0_ref[...]
        d1 = a1 - mu1_ref[...]
        w = jnp.exp(-0.5 * (d0 * d0 * r0 + d1 * d1 * r1))
        acc = w[:, 0:1] * h[:, 0:co]
        for kk in range(1, K):
            acc = acc + w[:, kk:kk + 1] * h[:, kk * co:(kk + 1) * co]
        o_ref[...] = jnp.concatenate(
            [acc, jnp.full((BE, 1), 1.0, jnp.float32),
             jnp.zeros((BE, 127 - co), jnp.float32)], axis=1)

    return pl.pallas_call(
        body,
        grid=(E // BE,),
        in_specs=[
            pl.BlockSpec((BE, 128), lambda i: (i, 0)),
            pl.BlockSpec((BE, 2), lambda i: (i, 0)),
            pl.BlockSpec((ci, K * co), lambda i: (0, 0)),
            pl.BlockSpec((1, K), lambda i: (0, 0)),
            pl.BlockSpec((1, K), lambda i: (0, 0)),
            pl.BlockSpec((1, K), lambda i: (0, 0)),
            pl.BlockSpec((1, K), lambda i: (0, 0)),
        ],
        out_specs=pl.BlockSpec((BE, 128), lambda i: (i, 0)),
        out_shape=jax.ShapeDtypeStruct((E, 128), jnp.float32),
    )(xe, attr, g, mu0, mu1, s0, s1)


def _tc_epilogue(parts, x_in, root, bias, bn_g, bn_b, ci, co):
    def body(p_ref, x_ref, r_ref, b_ref, g_ref, be_ref, o_ref):
        ssum = p_ref[0, :, :co] + p_ref[1, :, :co]
        cnt = p_ref[0, :, co:co + 1] + p_ref[1, :, co:co + 1]
        agg = ssum / jnp.maximum(cnt, 1.0)
        o = agg + jnp.dot(x_ref[...][:, :ci], r_ref[...],
                          preferred_element_type=jnp.float32) + b_ref[...]
        m = jnp.mean(o, axis=0, keepdims=True)
        v = jnp.mean((o - m) * (o - m), axis=0, keepdims=True)
        o = (o - m) / jnp.sqrt(v + 1e-5) * g_ref[...] + be_ref[...]
        o = jnp.where(o > 0.0, o, jnp.exp(o) - 1.0)
        o_ref[...] = jnp.concatenate(
            [o, jnp.zeros((N, 128 - co), jnp.float32)], axis=1)

    return pl.pallas_call(
        body,
        out_shape=jax.ShapeDtypeStruct((N, 128), jnp.float32),
    )(parts, x_in, root, bias.reshape(1, co),
      bn_g.reshape(1, co), bn_b.reshape(1, co))


def _tc_final(h, batch2d, fc1_w, fc1_b, fc2_w, fc2_b):
    def body(h_ref, b_ref, w1_ref, b1_ref, w2_ref, b2_ref, o_ref):
        oh = (b_ref[...] == lax.broadcasted_iota(jnp.int32, (N, NG), 1))
        oh = oh.astype(jnp.float32)
        pooled = lax.dot_general(oh, h_ref[...][:, :64],
                                 (((0,), (0,)), ((), ())),
                                 preferred_element_type=jnp.float32)
        ones = jnp.full((N, 1), 1.0, jnp.float32)
        pcnt = lax.dot_general(oh, ones, (((0,), (0,)), ((), ())),
                               preferred_element_type=jnp.float32)
        pooled = pooled / jnp.maximum(pcnt, 1.0)
        t = jnp.dot(pooled, w1_ref[...],
                    preferred_element_type=jnp.float32) + b1_ref[...]
        t = jnp.where(t > 0.0, t, jnp.exp(t) - 1.0)
        t = jnp.dot(t, w2_ref[...],
                    preferred_element_type=jnp.float32) + b2_ref[...]
        t = jnp.where(t > 0.0, t, jnp.exp(t) - 1.0)
        m = jnp.max(t, axis=1, keepdims=True)
        s = jnp.log(jnp.sum(jnp.exp(t - m), axis=1, keepdims=True))
        o_ref[...] = t - m - s

    return pl.pallas_call(
        body,
        out_shape=jax.ShapeDtypeStruct((NG, 10), jnp.float32),
    )(h, batch2d, fc1_w, fc1_b.reshape(1, -1), fc2_w, fc2_b.reshape(1, -1))


def kernel(x, edge_index, edge_attr, batch,
           g1, mu1, sigma1, root1, bias1, bn1_g, bn1_b,
           g2, mu2, sigma2, root2, bias2, bn2_g, bn2_b,
           g3, mu3, sigma3, root3, bias3, bn3_g, bn3_b,
           fc1_w, fc1_b, fc2_w, fc2_b):
    src = edge_index[0]
    dst = edge_index[1]

    layers = [
        (g1, mu1, sigma1, root1, bias1, bn1_g, bn1_b, 128, 32),
        (g2, mu2, sigma2, root2, bias2, bn2_g, bn2_b, 32, 64),
        (g3, mu3, sigma3, root3, bias3, bn3_g, bn3_b, 64, 64),
    ]
    h = x
    for (g, mu, sigma, root, bias, bng, bnb, ci, co) in layers:
        xe = _sc_gather(h, src, 128)
        msg = _tc_edge(xe, edge_attr, g,
                       mu[:, 0].reshape(1, K), mu[:, 1].reshape(1, K),
                       sigma[:, 0].reshape(1, K), sigma[:, 1].reshape(1, K),
                       ci, co)
        parts = _sc_scatter_add(msg, dst, 128)
        h = _tc_epilogue(parts, h, root, bias, bng, bnb, ci, co)

    return _tc_final(h, batch.reshape(N, 1), fc1_w, fc1_b, fc2_w, fc2_b)


# speedup vs baseline: 2.1646x; 2.0185x over previous
import functools

import jax
import jax.numpy as jnp
from jax import lax
from jax.experimental import pallas as pl
from jax.experimental.pallas import tpu as pltpu
from jax.experimental.pallas import tpu_sc as plsc

N = 10000
E = 320000
K = 25
NG = 128

NC = 2
NS = 16
NW = NC * NS
EW = E // NW
CH = 80
NCH = EW // CH

_MESH = dict(core_axis_name="c", subcore_axis_name="s")


def _sc_gather(table, idx, d):
    @functools.partial(
        pl.kernel,
        mesh=plsc.VectorSubcoreMesh(**_MESH),
        out_type=jax.ShapeDtypeStruct((E, d), jnp.float32),
        scratch_types=[
            pltpu.VMEM((CH,), jnp.int32),
            pltpu.VMEM((CH, d), jnp.float32),
            pltpu.SemaphoreType.DMA,
        ],
    )
    def k(table_hbm, idx_hbm, out_hbm, idx_v, rows_v, sem):
        wid = lax.axis_index("s") * NC + lax.axis_index("c")
        base = wid * EW

        @pl.loop(0, NCH)
        def _(c):
            off = base + c * CH
            pltpu.sync_copy(idx_hbm.at[pl.ds(off, CH)], idx_v)
            pltpu.async_copy(table_hbm.at[idx_v], rows_v, sem).wait()
            pltpu.sync_copy(rows_v, out_hbm.at[pl.ds(off, CH)])

    return k(table, idx)


def _sc_scatter_add(vals, idx, d):
    @functools.partial(
        pl.kernel,
        mesh=plsc.VectorSubcoreMesh(**_MESH),
        out_type=jax.ShapeDtypeStruct((NC, N, d), jnp.float32),
        scratch_types=[
            pltpu.VMEM_SHARED((N, d), jnp.float32),
            pltpu.VMEM((CH,), jnp.int32),
            pltpu.VMEM((CH, d), jnp.float32),
            pltpu.SemaphoreType.DMA,
        ],
    )
    def k(vals_hbm, idx_hbm, zero_hbm, out_hbm, acc_s, idx_v, val_v, sem):
        cid = lax.axis_index("c")
        sid = lax.axis_index("s")

        @pl.when(sid == 0)
        def _():
            pltpu.sync_copy(zero_hbm, acc_s)

        plsc.subcore_barrier()
        base = (sid * NC + cid) * EW

        @pl.loop(0, NCH)
        def _(c):
            off = base + c * CH
            pltpu.sync_copy(idx_hbm.at[pl.ds(off, CH)], idx_v)
            pltpu.sync_copy(vals_hbm.at[pl.ds(off, CH)], val_v)
            pltpu.sync_copy(val_v, acc_s.at[idx_v], add=True)

        plsc.subcore_barrier()

        @pl.when(sid == 0)
        def _():
            pltpu.sync_copy(acc_s, out_hbm.at[cid])

    zeros = jnp.zeros((N, d), jnp.float32)
    return k(vals, idx, zeros)


BE = 512


def _tc_edge(xe, attr, g, mu0, mu1, s0, s1, ci, co):
    kper = 128 // co
    kp = ((K + kper - 1) // kper) * kper
    ngrp = kp * co // 128
    gp = jnp.concatenate(
        [g, jnp.zeros((ci, (kp - K) * co), jnp.float32)], axis=1)
    rmat = jnp.kron(jnp.eye(kp, dtype=jnp.float32)[:K, :],
                    jnp.ones((1, co), jnp.float32))

    def body(xe_ref, attr_ref, g_ref, r_ref, mu0_ref, mu1_ref, s0_ref,
             s1_ref, o_ref):
        h = jnp.dot(xe_ref[...][:, :ci], g_ref[...],
                    preferred_element_type=jnp.float32)
        a = attr_ref[...]
        a0 = a[:, 0:1]
        a1 = a[:, 1:2]
        r0 = 1.0 / (1e-14 + s0_ref[...] * s0_ref[...])
        r1 = 1.0 / (1e-14 + s1_ref[...] * s1_ref[...])
        d0 = a0 - mu0_ref[...]
        d1 = a1 - mu1_ref[...]
        w = jnp.exp(-0.5 * (d0 * d0 * r0 + d1 * d1 * r1))
        wrep = jnp.dot(w, r_ref[...], preferred_element_type=jnp.float32)
        acc = h[:, 0:128] * wrep[:, 0:128]
        for j in range(1, ngrp):
            acc = acc + h[:, j * 128:(j + 1) * 128] * \
                wrep[:, j * 128:(j + 1) * 128]
        width = 128
        while width > co:
            width //= 2
            acc = acc[:, :width] + acc[:, width:]
        o_ref[...] = jnp.concatenate(
            [acc, jnp.full((BE, 1), 1.0, jnp.float32),
             jnp.zeros((BE, 127 - co), jnp.float32)], axis=1)

    return pl.pallas_call(
        body,
        grid=(E // BE,),
        in_specs=[
            pl.BlockSpec((BE, 128), lambda i: (i, 0)),
            pl.BlockSpec((BE, 2), lambda i: (i, 0)),
            pl.BlockSpec((ci, kp * co), lambda i: (0, 0)),
            pl.BlockSpec((K, kp * co), lambda i: (0, 0)),
            pl.BlockSpec((1, K), lambda i: (0, 0)),
            pl.BlockSpec((1, K), lambda i: (0, 0)),
            pl.BlockSpec((1, K), lambda i: (0, 0)),
            pl.BlockSpec((1, K), lambda i: (0, 0)),
        ],
        out_specs=pl.BlockSpec((BE, 128), lambda i: (i, 0)),
        out_shape=jax.ShapeDtypeStruct((E, 128), jnp.float32),
    )(xe, attr, gp, rmat, mu0, mu1, s0, s1)


def _tc_epilogue(parts, x_in, root, bias, bn_g, bn_b, ci, co):
    def body(p_ref, x_ref, r_ref, b_ref, g_ref, be_ref, o_ref):
        ssum = p_ref[0, :, :co] + p_ref[1, :, :co]
        cnt = p_ref[0, :, co:co + 1] + p_ref[1, :, co:co + 1]
        agg = ssum / jnp.maximum(cnt, 1.0)
        o = agg + jnp.dot(x_ref[...][:, :ci], r_ref[...],
                          preferred_element_type=jnp.float32) + b_ref[...]
        m = jnp.mean(o, axis=0, keepdims=True)
        v = jnp.mean((o - m) * (o - m), axis=0, keepdims=True)
        o = (o - m) / jnp.sqrt(v + 1e-5) * g_ref[...] + be_ref[...]
        o = jnp.where(o > 0.0, o, jnp.exp(o) - 1.0)
        o_ref[...] = jnp.concatenate(
            [o, jnp.zeros((N, 128 - co), jnp.float32)], axis=1)

    return pl.pallas_call(
        body,
        out_shape=jax.ShapeDtypeStruct((N, 128), jnp.float32),
    )(parts, x_in, root, bias.reshape(1, co),
      bn_g.reshape(1, co), bn_b.reshape(1, co))


def _tc_final(h, batch2d, fc1_w, fc1_b, fc2_w, fc2_b):
    def body(h_ref, b_ref, w1_ref, b1_ref, w2_ref, b2_ref, o_ref):
        oh = (b_ref[...] == lax.broadcasted_iota(jnp.int32, (N, NG), 1))
        oh = oh.astype(jnp.float32)
        pooled = lax.dot_general(oh, h_ref[...][:, :64],
                                 (((0,), (0,)), ((), ())),
                                 preferred_element_type=jnp.float32)
        ones = jnp.full((N, 1), 1.0, jnp.float32)
        pcnt = lax.dot_general(oh, ones, (((0,), (0,)), ((), ())),
                               preferred_element_type=jnp.float32)
        pooled = pooled / jnp.maximum(pcnt, 1.0)
        t = jnp.dot(pooled, w1_ref[...],
                    preferred_element_type=jnp.float32) + b1_ref[...]
        t = jnp.where(t > 0.0, t, jnp.exp(t) - 1.0)
        t = jnp.dot(t, w2_ref[...],
                    preferred_element_type=jnp.float32) + b2_ref[...]
        t = jnp.where(t > 0.0, t, jnp.exp(t) - 1.0)
        m = jnp.max(t, axis=1, keepdims=True)
        s = jnp.log(jnp.sum(jnp.exp(t - m), axis=1, keepdims=True))
        o_ref[...] = t - m - s

    return pl.pallas_call(
        body,
        out_shape=jax.ShapeDtypeStruct((NG, 10), jnp.float32),
    )(h, batch2d, fc1_w, fc1_b.reshape(1, -1), fc2_w, fc2_b.reshape(1, -1))


def kernel(x, edge_index, edge_attr, batch,
           g1, mu1, sigma1, root1, bias1, bn1_g, bn1_b,
           g2, mu2, sigma2, root2, bias2, bn2_g, bn2_b,
           g3, mu3, sigma3, root3, bias3, bn3_g, bn3_b,
           fc1_w, fc1_b, fc2_w, fc2_b):
    src = edge_index[0]
    dst = edge_index[1]

    layers = [
        (g1, mu1, sigma1, root1, bias1, bn1_g, bn1_b, 128, 32),
        (g2, mu2, sigma2, root2, bias2, bn2_g, bn2_b, 32, 64),
        (g3, mu3, sigma3, root3, bias3, bn3_g, bn3_b, 64, 64),
    ]
    h = x
    for (g, mu, sigma, root, bias, bng, bnb, ci, co) in layers:
        xe = _sc_gather(h, src, 128)
        msg = _tc_edge(xe, edge_attr, g,
                       mu[:, 0].reshape(1, K), mu[:, 1].reshape(1, K),
                       sigma[:, 0].reshape(1, K), sigma[:, 1].reshape(1, K),
                       ci, co)
        parts = _sc_scatter_add(msg, dst, 128)
        h = _tc_epilogue(parts, h, root, bias, bng, bnb, ci, co)

    return _tc_final(h, batch.reshape(N, 1), fc1_w, fc1_b, fc2_w, fc2_b)


# speedup vs baseline: 2.5609x; 1.1830x over previous
import functools

import jax
import jax.numpy as jnp
from jax import lax
from jax.experimental import pallas as pl
from jax.experimental.pallas import tpu as pltpu
from jax.experimental.pallas import tpu_sc as plsc

N = 10000
E = 320000
K = 25
NG = 128

NC = 2
NS = 16
NW = NC * NS
EW = E // NW
CH = 40
NCH = EW // CH

_MESH = dict(core_axis_name="c", subcore_axis_name="s")


GRP = 5
SGRP = 3


def _sc_gather(table, idx2d, d):
    @functools.partial(
        pl.kernel,
        mesh=plsc.VectorSubcoreMesh(**_MESH),
        out_type=jax.ShapeDtypeStruct((E, d), jnp.float32),
        scratch_types=[
            pltpu.VMEM((NCH, CH), jnp.int32),
            pltpu.VMEM((2 * GRP, CH, 128), jnp.float32),
            pltpu.SemaphoreType.DMA,
            pltpu.SemaphoreType.DMA,
        ],
    )
    def k(table_hbm, idx_hbm, out_hbm, idx_v, rows_v, sem_g, sem_s):
        wid = lax.axis_index("s") * NC + lax.axis_index("c")
        base = wid * EW
        pltpu.sync_copy(idx_hbm.at[wid], idx_v)

        def fire_gathers(c, lo):
            return [pltpu.async_copy(
                table_hbm.at[idx_v.at[c + lo + b]],
                rows_v.at[lo + b], sem_g) for b in range(GRP)]

        def fire_stores(c, lo):
            return [pltpu.async_copy(
                rows_v.at[lo + b],
                out_hbm.at[pl.ds(base + (c + lo + b) * CH, CH)], sem_s)
                for b in range(GRP)]

        @pl.loop(0, NCH, step=2 * GRP)
        def _(c):
            ga = fire_gathers(c, 0)
            for h in ga:
                h.wait()
            sa = fire_stores(c, 0)
            gb = fire_gathers(c, GRP)
            for h in gb:
                h.wait()
            sb = fire_stores(c, GRP)
            for h in sa + sb:
                h.wait()

    return k(table, idx2d)


def _sc_scatter_add(vals, idx2d, d):
    @functools.partial(
        pl.kernel,
        mesh=plsc.VectorSubcoreMesh(**_MESH),
        out_type=jax.ShapeDtypeStruct((NC, N, d), jnp.float32),
        scratch_types=[
            pltpu.VMEM_SHARED((N, d), jnp.float32),
            pltpu.VMEM((NCH, CH), jnp.int32),
            pltpu.VMEM((SGRP, CH, 128), jnp.float32),
            pltpu.SemaphoreType.DMA,
            pltpu.SemaphoreType.DMA,
        ],
    )
    def k(vals_hbm, idx_hbm, zero_hbm, out_hbm, acc_s, idx_v, val_v,
          sem_l, sem_a):
        cid = lax.axis_index("c")
        sid = lax.axis_index("s")

        @pl.when(sid == 0)
        def _():
            pltpu.sync_copy(zero_hbm, acc_s)

        wid = sid * NC + cid
        base = wid * EW
        pltpu.sync_copy(idx_hbm.at[wid], idx_v)
        plsc.subcore_barrier()

        @pl.loop(0, NCH - NCH % SGRP, step=SGRP)
        def _(c):
            loads = [pltpu.async_copy(
                vals_hbm.at[pl.ds(base + (c + b) * CH, CH)],
                val_v.at[b], sem_l) for b in range(SGRP)]
            for h in loads:
                h.wait()
            adds = [pltpu.async_copy(
                val_v.at[b],
                acc_s.at[idx_v.at[c + b]], sem_a, add=True)
                for b in range(SGRP)]
            for h in adds:
                h.wait()

        for cc in range(NCH - NCH % SGRP, NCH):
            pltpu.async_copy(
                vals_hbm.at[pl.ds(base + cc * CH, CH)],
                val_v.at[0], sem_l).wait()
            pltpu.async_copy(
                val_v.at[0], acc_s.at[idx_v.at[cc]], sem_a,
                add=True).wait()

        plsc.subcore_barrier()

        @pl.when(sid == 0)
        def _():
            pltpu.sync_copy(acc_s, out_hbm.at[cid])

    zeros = jnp.zeros((N, d), jnp.float32)
    return k(vals, idx2d, zeros)


BE = 512


def _tc_edge(xe, attr, g, mu0, mu1, s0, s1, ci, co):
    kper = 128 // co
    kp = ((K + kper - 1) // kper) * kper
    ngrp = kp * co // 128
    gp = jnp.concatenate(
        [g, jnp.zeros((ci, (kp - K) * co), jnp.float32)], axis=1)
    rmat = jnp.kron(jnp.eye(kp, dtype=jnp.float32)[:K, :],
                    jnp.ones((1, co), jnp.float32))

    def body(xe_ref, attr_ref, g_ref, r_ref, mu0_ref, mu1_ref, s0_ref,
             s1_ref, o_ref):
        h = jnp.dot(xe_ref[...][:, :ci], g_ref[...],
                    preferred_element_type=jnp.float32)
        a = attr_ref[...]
        a0 = a[:, 0:1]
        a1 = a[:, 1:2]
        r0 = 1.0 / (1e-14 + s0_ref[...] * s0_ref[...])
        r1 = 1.0 / (1e-14 + s1_ref[...] * s1_ref[...])
        d0 = a0 - mu0_ref[...]
        d1 = a1 - mu1_ref[...]
        w = jnp.exp(-0.5 * (d0 * d0 * r0 + d1 * d1 * r1))
        wrep = jnp.dot(w, r_ref[...], preferred_element_type=jnp.float32)
        acc = h[:, 0:128] * wrep[:, 0:128]
        for j in range(1, ngrp):
            acc = acc + h[:, j * 128:(j + 1) * 128] * \
                wrep[:, j * 128:(j + 1) * 128]
        width = 128
        while width > co:
            width //= 2
            acc = acc[:, :width] + acc[:, width:]
        o_ref[...] = jnp.concatenate(
            [acc, jnp.full((BE, 1), 1.0, jnp.float32),
             jnp.zeros((BE, 127 - co), jnp.float32)], axis=1)

    return pl.pallas_call(
        body,
        grid=(E // BE,),
        in_specs=[
            pl.BlockSpec((BE, 128), lambda i: (i, 0)),
            pl.BlockSpec((BE, 2), lambda i: (i, 0)),
            pl.BlockSpec((ci, kp * co), lambda i: (0, 0)),
            pl.BlockSpec((K, kp * co), lambda i: (0, 0)),
            pl.BlockSpec((1, K), lambda i: (0, 0)),
            pl.BlockSpec((1, K), lambda i: (0, 0)),
            pl.BlockSpec((1, K), lambda i: (0, 0)),
            pl.BlockSpec((1, K), lambda i: (0, 0)),
        ],
        out_specs=pl.BlockSpec((BE, 128), lambda i: (i, 0)),
        out_shape=jax.ShapeDtypeStruct((E, 128), jnp.float32),
    )(xe, attr, gp, rmat, mu0, mu1, s0, s1)


def _tc_epilogue(parts, x_in, root, bias, bn_g, bn_b, ci, co):
    def body(p_ref, x_ref, r_ref, b_ref, g_ref, be_ref, o_ref):
        ssum = p_ref[0, :, :co] + p_ref[1, :, :co]
        cnt = p_ref[0, :, co:co + 1] + p_ref[1, :, co:co + 1]
        agg = ssum / jnp.maximum(cnt, 1.0)
        o = agg + jnp.dot(x_ref[...][:, :ci], r_ref[...],
                          preferred_element_type=jnp.float32) + b_ref[...]
        m = jnp.mean(o, axis=0, keepdims=True)
        v = jnp.mean((o - m) * (o - m), axis=0, keepdims=True)
        o = (o - m) / jnp.sqrt(v + 1e-5) * g_ref[...] + be_ref[...]
        o = jnp.where(o > 0.0, o, jnp.exp(o) - 1.0)
        o_ref[...] = jnp.concatenate(
            [o, jnp.zeros((N, 128 - co), jnp.float32)], axis=1)

    return pl.pallas_call(
        body,
        out_shape=jax.ShapeDtypeStruct((N, 128), jnp.float32),
    )(parts, x_in, root, bias.reshape(1, co),
      bn_g.reshape(1, co), bn_b.reshape(1, co))


def _tc_final(h, batch2d, fc1_w, fc1_b, fc2_w, fc2_b):
    def body(h_ref, b_ref, w1_ref, b1_ref, w2_ref, b2_ref, o_ref):
        oh = (b_ref[...] == lax.broadcasted_iota(jnp.int32, (N, NG), 1))
        oh = oh.astype(jnp.float32)
        pooled = lax.dot_general(oh, h_ref[...][:, :64],
                                 (((0,), (0,)), ((), ())),
                                 preferred_element_type=jnp.float32)
        ones = jnp.full((N, 1), 1.0, jnp.float32)
        pcnt = lax.dot_general(oh, ones, (((0,), (0,)), ((), ())),
                               preferred_element_type=jnp.float32)
        pooled = pooled / jnp.maximum(pcnt, 1.0)
        t = jnp.dot(pooled, w1_ref[...],
                    preferred_element_type=jnp.float32) + b1_ref[...]
        t = jnp.where(t > 0.0, t, jnp.exp(t) - 1.0)
        t = jnp.dot(t, w2_ref[...],
                    preferred_element_type=jnp.float32) + b2_ref[...]
        t = jnp.where(t > 0.0, t, jnp.exp(t) - 1.0)
        m = jnp.max(t, axis=1, keepdims=True)
        s = jnp.log(jnp.sum(jnp.exp(t - m), axis=1, keepdims=True))
        o_ref[...] = t - m - s

    return pl.pallas_call(
        body,
        out_shape=jax.ShapeDtypeStruct((NG, 10), jnp.float32),
    )(h, batch2d, fc1_w, fc1_b.reshape(1, -1), fc2_w, fc2_b.reshape(1, -1))


def kernel(x, edge_index, edge_attr, batch,
           g1, mu1, sigma1, root1, bias1, bn1_g, bn1_b,
           g2, mu2, sigma2, root2, bias2, bn2_g, bn2_b,
           g3, mu3, sigma3, root3, bias3, bn3_g, bn3_b,
           fc1_w, fc1_b, fc2_w, fc2_b):
    src2d = edge_index[0].reshape(NW, NCH, CH)
    dst2d = edge_index[1].reshape(NW, NCH, CH)

    layers = [
        (g1, mu1, sigma1, root1, bias1, bn1_g, bn1_b, 128, 32),
        (g2, mu2, sigma2, root2, bias2, bn2_g, bn2_b, 32, 64),
        (g3, mu3, sigma3, root3, bias3, bn3_g, bn3_b, 64, 64),
    ]
    h = x
    for (g, mu, sigma, root, bias, bng, bnb, ci, co) in layers:
        xe = _sc_gather(h, src2d, 128)
        msg = _tc_edge(xe, edge_attr, g,
                       mu[:, 0].reshape(1, K), mu[:, 1].reshape(1, K),
                       sigma[:, 0].reshape(1, K), sigma[:, 1].reshape(1, K),
                       ci, co)
        parts = _sc_scatter_add(msg, dst2d, 128)
        h = _tc_epilogue(parts, h, root, bias, bng, bnb, ci, co)

    return _tc_final(h, batch.reshape(N, 1), fc1_w, fc1_b, fc2_w, fc2_b)
